# Initial kernel scaffold; baseline (speedup 1.0000x reference)
import functools

import jax
import jax.numpy as jnp
from jax import lax
from jax.experimental import pallas as pl
from jax.experimental.pallas import tpu as pltpu


def _fps_body(px_ref, py_ref, pz_ref, ox_ref, oy_ref, oz_ref, *, npoint, n):
    b = px_ref.shape[0]
    px = px_ref[...]
    py = py_ref[...]
    pz = pz_ref[...]
    flat = lax.broadcasted_iota(jnp.int32, (b, n), 1)
    ninf = jnp.float32(-jnp.inf)

    def body(i, carry):
        dists, far = carry
        sel = flat == far
        cx = jnp.max(jnp.where(sel, px, ninf), axis=1, keepdims=True)
        cy = jnp.max(jnp.where(sel, py, ninf), axis=1, keepdims=True)
        cz = jnp.max(jnp.where(sel, pz, ninf), axis=1, keepdims=True)
        ox_ref[:, pl.ds(i, 1)] = cx
        oy_ref[:, pl.ds(i, 1)] = cy
        oz_ref[:, pl.ds(i, 1)] = cz
        dx = px - cx
        dy = py - cy
        dz = pz - cz
        d = (dx * dx + dy * dy) + dz * dz
        dists = jnp.minimum(dists, d)
        mx = jnp.max(dists, axis=1, keepdims=True)
        far = jnp.min(jnp.where(dists == mx, flat, n), axis=1, keepdims=True)
        return dists, far

    dists0 = jnp.full((b, n), 1e10, dtype=jnp.float32)
    far0 = jnp.zeros((b, 1), dtype=jnp.int32)
    lax.fori_loop(0, npoint, body, (dists0, far0))


def _fps_call(px, py, pz, npoint):
    b, n = px.shape
    out_shape = [jax.ShapeDtypeStruct((b, npoint), jnp.float32)] * 3
    fn = pl.pallas_call(
        functools.partial(_fps_body, npoint=npoint, n=n),
        out_shape=out_shape,
    )
    return fn(px, py, pz)


def _conv_body(centers_ref, post_ref, table_ref, *refs, r2, n, blk, ctab):
    out_ref = refs[-1]
    w_refs = refs[:-1]
    cx = centers_ref[0, :, 0:1]
    cy = centers_ref[0, :, 1:2]
    cz = centers_ref[0, :, 2:3]
    px = post_ref[0, 0:1, :]
    py = post_ref[0, 1:2, :]
    pz = post_ref[0, 2:3, :]
    dx = cx - px
    dy = cy - py
    dz = cz - pz
    d2 = (dx * dx + dy * dy) + dz * dz
    iota = lax.broadcasted_iota(jnp.int32, (blk, n), 1)
    midx = jnp.where(d2 > r2, n, iota)
    idx1 = jnp.min(midx, axis=1, keepdims=True)
    idx2 = jnp.min(jnp.where(midx > idx1, midx, n), axis=1, keepdims=True)
    idx2 = jnp.where(idx2 == n, idx1, idx2)
    table = table_ref[0]
    oh1 = (iota == idx1).astype(jnp.float32)
    oh2 = (iota == idx2).astype(jnp.float32)
    row1 = jnp.dot(oh1, table, preferred_element_type=jnp.float32)
    row2 = jnp.dot(oh2, table, preferred_element_type=jnp.float32)
    cpad = jnp.concatenate(
        [centers_ref[0], jnp.zeros((blk, ctab - 3), jnp.float32)], axis=1)
    g = jnp.concatenate([row1 - cpad, row2 - cpad], axis=0)
    for i in range(0, len(w_refs), 2):
        w = w_refs[i][...]
        bias = w_refs[i + 1][...]
        g = jnp.maximum(jnp.dot(g, w, preferred_element_type=jnp.float32) + bias, 0.0)
    out_ref[0] = jnp.maximum(g[:blk], g[blk:])


def _conv_call(centers, post, table, layers, radius, blk):
    b, npoint, _ = centers.shape
    n = post.shape[2]
    ctab = table.shape[2]
    cout = layers[-1][0].shape[1]
    w_args = []
    w_specs = []
    for (w, bias) in layers:
        w_args.append(w)
        w_specs.append(pl.BlockSpec(w.shape, lambda bb, j: (0, 0)))
        bias2 = bias.reshape(1, -1)
        w_args.append(bias2)
        w_specs.append(pl.BlockSpec(bias2.shape, lambda bb, j: (0, 0)))
    fn = pl.pallas_call(
        functools.partial(_conv_body, r2=radius * radius, n=n, blk=blk, ctab=ctab),
        grid=(b, npoint // blk),
        in_specs=[
            pl.BlockSpec((1, blk, 3), lambda bb, j: (bb, j, 0)),
            pl.BlockSpec((1, 3, n), lambda bb, j: (bb, 0, 0)),
            pl.BlockSpec((1, n, ctab), lambda bb, j: (bb, 0, 0)),
            *w_specs,
        ],
        out_specs=pl.BlockSpec((1, blk, cout), lambda bb, j: (bb, j, 0)),
        out_shape=jax.ShapeDtypeStruct((b, npoint, cout), jnp.float32),
    )
    return fn(centers, post, table, *w_args)


def kernel(x, features, params):
    pos = x[:, :, :3]
    feat = jnp.transpose(features, (0, 2, 1))

    px, py, pz = pos[:, :, 0], pos[:, :, 1], pos[:, :, 2]
    nx1, ny1, nz1 = _fps_call(px, py, pz, npoint=2048)
    pos2 = jnp.stack([nx1, ny1, nz1], axis=-1)
    post1 = jnp.transpose(pos, (0, 2, 1))
    table1 = jnp.concatenate([pos, features], axis=-1)
    nf1 = _conv_call(pos2, post1, table1, params["l1"], radius=0.5, blk=128)
    feat2 = jnp.transpose(nf1, (0, 2, 1))

    nx2, ny2, nz2 = _fps_call(nx1, ny1, nz1, npoint=512)
    pos3 = jnp.stack([nx2, ny2, nz2], axis=-1)
    post2 = jnp.transpose(pos2, (0, 2, 1))
    table2 = jnp.concatenate([pos2, nf1], axis=-1)
    nf2 = _conv_call(pos3, post2, table2, params["l2"], radius=1.0, blk=128)
    feat3 = jnp.transpose(nf2, (0, 2, 1))

    return (pos, feat, pos2, feat2, pos3, feat3)


# speedup vs baseline: 22.0552x; 22.0552x over previous
import functools

import jax
import jax.numpy as jnp
from jax import lax
from jax.experimental import pallas as pl
from jax.experimental.pallas import tpu as pltpu


def _fps_body(px_ref, py_ref, pz_ref, out_ref, *, npoint, n):
    b = px_ref.shape[0]
    px = px_ref[...]
    py = py_ref[...]
    pz = pz_ref[...]
    flat = lax.broadcasted_iota(jnp.int32, (b, n), 1)
    ninf = jnp.float32(-jnp.inf)

    def body(i, carry):
        dists, far = carry
        sel = flat == far
        cx = jnp.max(jnp.where(sel, px, ninf), axis=1, keepdims=True)
        cy = jnp.max(jnp.where(sel, py, ninf), axis=1, keepdims=True)
        cz = jnp.max(jnp.where(sel, pz, ninf), axis=1, keepdims=True)
        c = jnp.concatenate([cx, cy, cz], axis=1)
        out_ref[pl.ds(i, 1)] = c[None]
        dx = px - cx
        dy = py - cy
        dz = pz - cz
        d = (dx * dx + dy * dy) + dz * dz
        dists = jnp.minimum(dists, d)
        mx = jnp.max(dists, axis=1, keepdims=True)
        far = jnp.min(jnp.where(dists == mx, flat, n), axis=1, keepdims=True)
        return dists, far

    dists0 = jnp.full((b, n), 1e10, dtype=jnp.float32)
    far0 = jnp.zeros((b, 1), dtype=jnp.int32)
    lax.fori_loop(0, npoint, body, (dists0, far0))


def _fps_call(px, py, pz, npoint):
    b, n = px.shape
    fn = pl.pallas_call(
        functools.partial(_fps_body, npoint=npoint, n=n),
        out_shape=jax.ShapeDtypeStruct((npoint, b, 3), jnp.float32),
    )
    return jnp.transpose(fn(px, py, pz), (1, 0, 2))


def _conv_body(centers_ref, post_ref, table_ref, *refs, r2, n, blk, ctab):
    out_ref = refs[-1]
    w_refs = refs[:-1]
    cx = centers_ref[0, :, 0:1]
    cy = centers_ref[0, :, 1:2]
    cz = centers_ref[0, :, 2:3]
    px = post_ref[0, 0:1, :]
    py = post_ref[0, 1:2, :]
    pz = post_ref[0, 2:3, :]
    dx = cx - px
    dy = cy - py
    dz = cz - pz
    d2 = (dx * dx + dy * dy) + dz * dz
    iota = lax.broadcasted_iota(jnp.int32, (blk, n), 1)
    midx = jnp.where(d2 > r2, n, iota)
    idx1 = jnp.min(midx, axis=1, keepdims=True)
    idx2 = jnp.min(jnp.where(midx > idx1, midx, n), axis=1, keepdims=True)
    idx2 = jnp.where(idx2 == n, idx1, idx2)
    table = table_ref[0]
    oh1 = (iota == idx1).astype(jnp.float32)
    oh2 = (iota == idx2).astype(jnp.float32)
    row1 = jnp.dot(oh1, table, preferred_element_type=jnp.float32)
    row2 = jnp.dot(oh2, table, preferred_element_type=jnp.float32)
    cpad = jnp.concatenate(
        [centers_ref[0], jnp.zeros((blk, ctab - 3), jnp.float32)], axis=1)
    g = jnp.concatenate([row1 - cpad, row2 - cpad], axis=0)
    for i in range(0, len(w_refs), 2):
        w = w_refs[i][...]
        bias = w_refs[i + 1][...]
        g = jnp.maximum(jnp.dot(g, w, preferred_element_type=jnp.float32) + bias, 0.0)
    out_ref[0] = jnp.maximum(g[:blk], g[blk:])


def _conv_call(centers, post, table, layers, radius, blk):
    b, npoint, _ = centers.shape
    n = post.shape[2]
    ctab = table.shape[2]
    cout = layers[-1][0].shape[1]
    w_args = []
    w_specs = []
    for (w, bias) in layers:
        w_args.append(w)
        w_specs.append(pl.BlockSpec(w.shape, lambda bb, j: (0, 0)))
        bias2 = bias.reshape(1, -1)
        w_args.append(bias2)
        w_specs.append(pl.BlockSpec(bias2.shape, lambda bb, j: (0, 0)))
    fn = pl.pallas_call(
        functools.partial(_conv_body, r2=radius * radius, n=n, blk=blk, ctab=ctab),
        grid=(b, npoint // blk),
        in_specs=[
            pl.BlockSpec((1, blk, 3), lambda bb, j: (bb, j, 0)),
            pl.BlockSpec((1, 3, n), lambda bb, j: (bb, 0, 0)),
            pl.BlockSpec((1, n, ctab), lambda bb, j: (bb, 0, 0)),
            *w_specs,
        ],
        out_specs=pl.BlockSpec((1, blk, cout), lambda bb, j: (bb, j, 0)),
        out_shape=jax.ShapeDtypeStruct((b, npoint, cout), jnp.float32),
    )
    return fn(centers, post, table, *w_args)


def kernel(x, features, params):
    pos = x[:, :, :3]
    feat = jnp.transpose(features, (0, 2, 1))

    px, py, pz = pos[:, :, 0], pos[:, :, 1], pos[:, :, 2]
    pos2 = _fps_call(px, py, pz, npoint=2048)
    post1 = jnp.transpose(pos, (0, 2, 1))
    table1 = jnp.concatenate([pos, features], axis=-1)
    nf1 = _conv_call(pos2, post1, table1, params["l1"], radius=0.5, blk=128)
    feat2 = jnp.transpose(nf1, (0, 2, 1))

    pos3 = _fps_call(pos2[:, :, 0], pos2[:, :, 1], pos2[:, :, 2], npoint=512)
    post2 = jnp.transpose(pos2, (0, 2, 1))
    table2 = jnp.concatenate([pos2, nf1], axis=-1)
    nf2 = _conv_call(pos3, post2, table2, params["l2"], radius=1.0, blk=128)
    feat3 = jnp.transpose(nf2, (0, 2, 1))

    return (pos, feat, pos2, feat2, pos3, feat3)
